# baseline (speedup 1.0000x reference)
import functools

import jax
import jax.numpy as jnp
from jax import lax
from jax.experimental import pallas as pl
from jax.experimental.pallas import tpu as pltpu
from jax.experimental.pallas import tpu_sc as plsc

_SIZE = 512
_LENGTH = 16
_ATT = 256
_R = 1024
_LANES = 16
_W = 32


def _make_step(bsz):
    groups = bsz // _LANES
    mesh = plsc.VectorSubcoreMesh(core_axis_name="c", subcore_axis_name="s")

    @functools.partial(
        pl.kernel,
        out_type=jax.ShapeDtypeStruct((_SIZE, bsz), jnp.float32),
        mesh=mesh,
        compiler_params=pltpu.CompilerParams(use_tc_tiling_on_sc=False,
                                             needs_layout_passes=False),
        scratch_types=[
            pltpu.VMEM((_R + _LANES,), jnp.int32),
            pltpu.VMEM((_SIZE, _LANES), jnp.float32),
            pltpu.VMEM((_R, _LANES), jnp.float32),
            pltpu.VMEM((_SIZE, _LANES), jnp.float32),
            pltpu.SemaphoreType.DMA,
        ],
    )
    def step(packed_hbm, state_t_hbm, h_t_hbm, out_hbm,
             packed_v, state_v, h_v, out_v, sem):
        wid = lax.axis_index("s") * 2 + lax.axis_index("c")

        @pl.when(wid < groups)
        def _():
            base = wid * _LANES
            copies = [
                pltpu.async_copy(packed_hbm, packed_v.at[pl.ds(0, _R)], sem),
                pltpu.async_copy(state_t_hbm.at[:, pl.ds(base, _LANES)],
                                 state_v, sem),
                pltpu.async_copy(h_t_hbm.at[:, pl.ds(base, _LANES)],
                                 h_v, sem),
            ]
            zeros = jnp.zeros((_LANES,), jnp.float32)
            for s in range(_SIZE):
                out_v[s, :] = zeros
            for c in copies:
                c.wait()

            lanes = lax.iota(jnp.int32, _LANES)
            first_obj = packed_v[pl.ds(0, _LANES)][0] >> 10

            def splat(s):
                return jnp.zeros((_LANES,), jnp.int32) + s

            def body(j, carry):
                prev_obj, acc = carry
                pv = packed_v[pl.ds(j, _LANES)][0]
                oj = pv >> 10
                sj = pv & (_R - 1)
                val = (plsc.load_gather(state_v, [splat(sj), lanes]) *
                       plsc.load_gather(h_v, [splat(j), lanes]))
                boundary = (oj != prev_obj) | ((j > 0) & ((j & (_W - 1)) == 0))

                @pl.when(boundary)
                def _flush():
                    plsc.addupdate_scatter(out_v, [splat(prev_obj), lanes], acc)

                acc = jnp.where(boundary, val, acc + val)
                return oj, acc

            last_obj, acc = lax.fori_loop(
                0, _R, body, (first_obj, jnp.zeros((_LANES,), jnp.float32)))
            plsc.addupdate_scatter(out_v, [splat(last_obj), lanes], acc)
            pltpu.sync_copy(out_v, out_hbm.at[:, pl.ds(base, _LANES)])

    return step


def kernel(x, rel_subj, rel_obj, rel_enc, Wrel, brel, action_table, pos_table,
           metaMode_init, W1G, b1G, W2G, b2G):
    bsz = x.shape[0]
    step = _make_step(bsz)

    perm = jnp.argsort(rel_obj, stable=True).astype(jnp.int32)
    packed = ((rel_obj[perm] << 10) | rel_subj[perm]).astype(jnp.int32)

    state = x[:, :_SIZE].astype(jnp.float32)
    metaMode = jnp.broadcast_to(metaMode_init[None], (bsz, _ATT))
    relation = jnp.dot(rel_enc[:_R], Wrel) + brel
    outs = []
    for _ in range(_LENGTH):
        g_in = jnp.concatenate((state, metaMode), axis=1)
        metaMode = jax.nn.relu(jnp.dot(g_in, W1G) + b1G)
        metaMode = jnp.dot(metaMode, W2G) + b2G
        h = jax.nn.sigmoid(jnp.dot(metaMode, relation.T))
        state = step(packed, state.T, h[:, perm].T).T
        outs.append(state)
    return jnp.stack(outs, axis=1)
